# baseline (speedup 1.0000x reference)
import functools

import jax
import jax.numpy as jnp
from jax import lax
from jax.experimental import pallas as pl
from jax.experimental.pallas import tpu as pltpu
from jax.experimental.pallas import tpu_sc as plsc

_NC = 2
_NS = 16
_NW = _NC * _NS

_N = 10000
_E = 320000
_D = 128
_H = 128
_Q = 100000

_KE = 96
_EW = _E // _NW
_CE = -(-_EW // _KE)
_EWP = _CE * _KE
_KD = 80
_CD = _E // _NS // _KD
_QP = 102400
_KQ = 128
_CQ = _QP // _NW // _KQ


def _sc_mesh():
    return plsc.VectorSubcoreMesh(core_axis_name="c", subcore_axis_name="s")


_NDEG = 20480
_NAGG = 10240


def _sc_degrees(idx4, zrows, ones_k):
    rpt = _NAGG // _NS

    @functools.partial(
        pl.kernel,
        mesh=_sc_mesh(),
        out_type=jax.ShapeDtypeStruct((_NC * _NAGG, _D), jnp.float32),
        scratch_types=[
            pltpu.VMEM_SHARED((_NAGG, _D), jnp.float32),
            pltpu.VMEM((_CD, _KD), jnp.int32),
            pltpu.VMEM((_KD, _D), jnp.float32),
        ],
    )
    def deg_kernel(idx_hbm, z_hbm, ones_hbm, out_hbm, deg_sh, idx_v, ones_v):
        c = lax.axis_index("c")
        s = lax.axis_index("s")
        base = s * rpt
        pltpu.sync_copy(z_hbm, deg_sh.at[pl.ds(base, rpt)])
        pltpu.sync_copy(idx_hbm.at[c, s], idx_v)
        pltpu.sync_copy(ones_hbm, ones_v)
        plsc.subcore_barrier()

        @pl.loop(0, _CD)
        def _(j):
            pltpu.sync_copy(ones_v, deg_sh.at[idx_v.at[j]], add=True)

        plsc.subcore_barrier()
        pltpu.sync_copy(deg_sh.at[pl.ds(base, rpt)],
                        out_hbm.at[pl.ds(c * _NAGG + base, rpt)])

    return deg_kernel(idx4, zrows, ones_k)


def _sc_edge_agg(hs, src3, dst3, zrows):
    rpt = _NAGG // _NS

    @functools.partial(
        pl.kernel,
        mesh=_sc_mesh(),
        out_type=jax.ShapeDtypeStruct((_NC * _NAGG, _D), jnp.float32),
        scratch_types=[
            pltpu.VMEM_SHARED((_NAGG, _D), jnp.float32),
            pltpu.VMEM((_CE, _KE), jnp.int32),
            pltpu.VMEM((_CE, _KE), jnp.int32),
            pltpu.VMEM((_KE, _D), jnp.float32),
            pltpu.SemaphoreType.DMA,
        ],
    )
    def agg_kernel(hs_hbm, src_hbm, dst_hbm, z_hbm, out_hbm,
                   agg_sh, src_v, dst_v, rows_v, gsem):
        c = lax.axis_index("c")
        s = lax.axis_index("s")
        wid = s * _NC + c
        base = s * rpt
        pltpu.sync_copy(z_hbm, agg_sh.at[pl.ds(base, rpt)])
        pltpu.sync_copy(src_hbm.at[wid], src_v)
        pltpu.sync_copy(dst_hbm.at[wid], dst_v)
        plsc.subcore_barrier()

        @pl.loop(0, _CE)
        def _(j):
            pltpu.async_copy(hs_hbm.at[src_v.at[j]], rows_v, gsem).wait()
            pltpu.sync_copy(rows_v, agg_sh.at[dst_v.at[j]], add=True)

        plsc.subcore_barrier()
        pltpu.sync_copy(agg_sh.at[pl.ds(base, rpt)],
                        out_hbm.at[pl.ds(c * _NAGG + base, rpt)])

    return agg_kernel(hs, src3, dst3, zrows)


def _sc_decode_gather(tbl, u3, v3):
    per_w = _CQ * _KQ

    @functools.partial(
        pl.kernel,
        mesh=_sc_mesh(),
        out_type=(jax.ShapeDtypeStruct((_QP, _D), jnp.float32),
                  jax.ShapeDtypeStruct((_QP, _D), jnp.float32)),
        scratch_types=[
            pltpu.VMEM((_CQ, _KQ), jnp.int32),
            pltpu.VMEM((_CQ, _KQ), jnp.int32),
            pltpu.VMEM((_KQ, _D), jnp.float32),
            pltpu.VMEM((_KQ, _D), jnp.float32),
            pltpu.VMEM((_KQ, _D), jnp.float32),
            pltpu.VMEM((_KQ, _D), jnp.float32),
            pltpu.SemaphoreType.DMA,
            pltpu.SemaphoreType.DMA,
            pltpu.SemaphoreType.DMA,
            pltpu.SemaphoreType.DMA,
        ],
    )
    def dec_kernel(t_hbm, u_hbm, v_hbm, gu_hbm, gv_hbm,
                   u_v, v_v, ru_a, rv_a, ru_b, rv_b,
                   semu_a, semv_a, semu_b, semv_b):
        c = lax.axis_index("c")
        s = lax.axis_index("s")
        wid = s * _NC + c
        qbase = wid * per_w
        pltpu.sync_copy(u_hbm.at[wid], u_v)
        pltpu.sync_copy(v_hbm.at[wid], v_v)

        pltpu.async_copy(t_hbm.at[u_v.at[0]], ru_a, semu_a)
        pltpu.async_copy(t_hbm.at[v_v.at[0]], rv_a, semv_a)

        @pl.loop(0, _CQ // 2)
        def _(h):
            j = h * 2
            pltpu.make_async_copy(t_hbm.at[u_v.at[j]], ru_a, semu_a).wait()
            pltpu.make_async_copy(t_hbm.at[v_v.at[j]], rv_a, semv_a).wait()
            pltpu.async_copy(t_hbm.at[u_v.at[j + 1]], ru_b, semu_b)
            pltpu.async_copy(t_hbm.at[v_v.at[j + 1]], rv_b, semv_b)
            pltpu.sync_copy(ru_a, gu_hbm.at[pl.ds(qbase + j * _KQ, _KQ)])
            pltpu.sync_copy(rv_a, gv_hbm.at[pl.ds(qbase + j * _KQ, _KQ)])
            pltpu.make_async_copy(t_hbm.at[u_v.at[j + 1]], ru_b,
                                  semu_b).wait()
            pltpu.make_async_copy(t_hbm.at[v_v.at[j + 1]], rv_b,
                                  semv_b).wait()

            @pl.when(j + 2 < _CQ)
            def _():
                pltpu.async_copy(t_hbm.at[u_v.at[j + 2]], ru_a, semu_a)
                pltpu.async_copy(t_hbm.at[v_v.at[j + 2]], rv_a, semv_a)

            pltpu.sync_copy(ru_b,
                            gu_hbm.at[pl.ds(qbase + (j + 1) * _KQ, _KQ)])
            pltpu.sync_copy(rv_b,
                            gv_hbm.at[pl.ds(qbase + (j + 1) * _KQ, _KQ)])

        if _CQ % 2:
            j = _CQ - 1
            pltpu.make_async_copy(t_hbm.at[u_v.at[j]], ru_a, semu_a).wait()
            pltpu.make_async_copy(t_hbm.at[v_v.at[j]], rv_a, semv_a).wait()
            pltpu.sync_copy(ru_a, gu_hbm.at[pl.ds(qbase + j * _KQ, _KQ)])
            pltpu.sync_copy(rv_a, gv_hbm.at[pl.ds(qbase + j * _KQ, _KQ)])

    return dec_kernel(tbl, u3, v3)


_R = 400
_NB = _N // _R


def _norm_from(deg_blk):
    return lax.rsqrt(jnp.maximum(deg_blk[0, :, 0:1], 1.0))


def _prep_body(x_ref, dego_ref, o_ref):
    o_ref[...] = x_ref[...] * _norm_from(dego_ref[...])


def _tc_prep(x, degp):
    return pl.pallas_call(
        _prep_body,
        grid=(_NB,),
        in_specs=[pl.BlockSpec((_R, _D), lambda i: (i, 0)),
                  pl.BlockSpec((1, _R, _D), lambda i: (0, i, 0))],
        out_specs=pl.BlockSpec((_R, _D), lambda i: (i, 0)),
        out_shape=jax.ShapeDtypeStruct((_N, _D), jnp.float32),
    )(x, degp)


def _mid_body(p_ref, degi_ref, dego_ref, w_ref, b_ref, o_ref):
    agg = (p_ref[0] + p_ref[1]) * _norm_from(degi_ref[...])
    h = jnp.maximum(
        jnp.dot(agg, w_ref[...], preferred_element_type=jnp.float32)
        + b_ref[...], 0.0)
    o_ref[...] = h * _norm_from(dego_ref[...])


def _tc_mid(parts, degp, w1, b1):
    return pl.pallas_call(
        _mid_body,
        grid=(_NB,),
        in_specs=[pl.BlockSpec((_NC, _R, _D), lambda i: (0, i, 0)),
                  pl.BlockSpec((1, _R, _D), lambda i: (1, i, 0)),
                  pl.BlockSpec((1, _R, _D), lambda i: (0, i, 0)),
                  pl.BlockSpec((_D, _H), lambda i: (0, 0)),
                  pl.BlockSpec((1, _H), lambda i: (0, 0))],
        out_specs=pl.BlockSpec((_R, _D), lambda i: (i, 0)),
        out_shape=jax.ShapeDtypeStruct((_N, _D), jnp.float32),
    )(parts, degp, degp, w1, b1)


def _fin_body(p_ref, degi_ref, w2_ref, b2_ref, wa_ref, wb_ref, bc1_ref, t_ref):
    agg = (p_ref[0] + p_ref[1]) * _norm_from(degi_ref[...])
    z = (jnp.dot(agg, w2_ref[...], preferred_element_type=jnp.float32)
         + b2_ref[...])
    t_ref[0] = jnp.dot(z, wa_ref[...], preferred_element_type=jnp.float32)
    t_ref[1] = (jnp.dot(z, wb_ref[...], preferred_element_type=jnp.float32)
                + bc1_ref[...])


def _tc_final(parts, degp, w2, b2, wc1a, wc1b, bc1):
    return pl.pallas_call(
        _fin_body,
        grid=(_NB,),
        in_specs=[pl.BlockSpec((_NC, _R, _D), lambda i: (0, i, 0)),
                  pl.BlockSpec((1, _R, _D), lambda i: (1, i, 0)),
                  pl.BlockSpec((_D, _H), lambda i: (0, 0)),
                  pl.BlockSpec((1, _H), lambda i: (0, 0)),
                  pl.BlockSpec((_H, _H), lambda i: (0, 0)),
                  pl.BlockSpec((_H, _H), lambda i: (0, 0)),
                  pl.BlockSpec((1, _H), lambda i: (0, 0))],
        out_specs=pl.BlockSpec((2, _R, _D), lambda i: (0, i, 0)),
        out_shape=jax.ShapeDtypeStruct((2, _N, _D), jnp.float32),
    )(parts, degp, w2, b2, wc1a, wc1b, bc1)


_RQ = 2048


def _dec_body(gu_ref, gv_ref, w_ref, b_ref, o_ref):
    g = jnp.maximum(gu_ref[...] + gv_ref[...], 0.0)
    o_ref[...] = (jnp.sum(g * w_ref[...], axis=1, keepdims=True)
                  + b_ref[...])


def _tc_decode(gu, gv, wc2t, bc2):
    return pl.pallas_call(
        _dec_body,
        grid=(_QP // _RQ,),
        in_specs=[pl.BlockSpec((_RQ, _D), lambda i: (i, 0)),
                  pl.BlockSpec((_RQ, _D), lambda i: (i, 0)),
                  pl.BlockSpec((1, _D), lambda i: (0, 0)),
                  pl.BlockSpec((1, 1), lambda i: (0, 0))],
        out_specs=pl.BlockSpec((_RQ, 1), lambda i: (i, 0)),
        out_shape=jax.ShapeDtypeStruct((_QP, 1), jnp.float32),
    )(gu, gv, wc2t, bc2)


def kernel(x, edge_index, u, v, W1, b1, W2, b2, Wc1, bc1, Wc2, bc2):
    src = edge_index[0].astype(jnp.int32)
    dst = edge_index[1].astype(jnp.int32)

    didx = jnp.stack([src.reshape(_NS, _CD, _KD),
                      dst.reshape(_NS, _CD, _KD)])
    pad = _EWP - _EW
    src3 = jnp.concatenate(
        [src.reshape(_NW, _EW),
         jnp.full((_NW, pad), _N, jnp.int32)], axis=1
    ).reshape(_NW, _CE, _KE)
    dst3 = jnp.concatenate(
        [dst.reshape(_NW, _EW),
         jnp.full((_NW, pad), _NAGG - 8, jnp.int32)], axis=1
    ).reshape(_NW, _CE, _KE)
    u3 = jnp.pad(u.astype(jnp.int32), (0, _QP - _Q)).reshape(_NW, _CQ, _KQ)
    v3 = jnp.pad(v.astype(jnp.int32) + _N, (0, _QP - _Q),
                 constant_values=_N).reshape(_NW, _CQ, _KQ)

    ones_k = jnp.ones((_KD, _D), jnp.float32)
    zagg = jnp.zeros((_NAGG // _NS, _D), jnp.float32)

    degp = _sc_degrees(didx, zagg, ones_k).reshape(_NC, _NAGG, _D)

    zpad = jnp.zeros((16, _D), jnp.float32)
    hs1 = jnp.concatenate([_tc_prep(x, degp), zpad])
    parts1 = _sc_edge_agg(hs1, src3, dst3, zagg).reshape(_NC, _NAGG, _D)
    hs2 = jnp.concatenate([_tc_mid(parts1, degp, W1, b1.reshape(1, _H)), zpad])
    parts2 = _sc_edge_agg(hs2, src3, dst3, zagg).reshape(_NC, _NAGG, _D)
    tbl = _tc_final(parts2, degp, W2, b2.reshape(1, _H),
                    Wc1[:_H], Wc1[_H:], bc1.reshape(1, _H)).reshape(2 * _N, _D)

    gu, gv = _sc_decode_gather(tbl, u3, v3)
    out = _tc_decode(gu, gv, Wc2.reshape(1, _H), bc2.reshape(1, 1))
    return out.reshape(_QP)[:_Q]


# speedup vs baseline: 1.1669x; 1.1669x over previous
import functools

import jax
import jax.numpy as jnp
from jax import lax
from jax.experimental import pallas as pl
from jax.experimental.pallas import tpu as pltpu
from jax.experimental.pallas import tpu_sc as plsc

_NC = 2
_NS = 16
_NW = _NC * _NS

_N = 10000
_E = 320000
_D = 128
_H = 128
_Q = 100000

_KE = 80
_EW = _E // _NW
_CE = -(-_EW // _KE)
_EWP = _CE * _KE
_KD = 80
_CD = _E // _NS // _KD
_QP = 102400
_KQ = 80
_CQ = _QP // _NW // _KQ


def _sc_mesh():
    return plsc.VectorSubcoreMesh(core_axis_name="c", subcore_axis_name="s")


_NDEG = 20480
_NAGG = 10240


def _sc_degrees(idx4, zrows, ones_k):
    rpt = _NAGG // _NS

    @functools.partial(
        pl.kernel,
        mesh=_sc_mesh(),
        out_type=jax.ShapeDtypeStruct((_NC * _NAGG, _D), jnp.float32),
        scratch_types=[
            pltpu.VMEM_SHARED((_NAGG, _D), jnp.float32),
            pltpu.VMEM((_CD, _KD), jnp.int32),
            pltpu.VMEM((_KD, _D), jnp.float32),
        ],
    )
    def deg_kernel(idx_hbm, z_hbm, ones_hbm, out_hbm, deg_sh, idx_v, ones_v):
        c = lax.axis_index("c")
        s = lax.axis_index("s")
        base = s * rpt
        pltpu.sync_copy(z_hbm, deg_sh.at[pl.ds(base, rpt)])
        pltpu.sync_copy(idx_hbm.at[c, s], idx_v)
        pltpu.sync_copy(ones_hbm, ones_v)
        plsc.subcore_barrier()

        @pl.loop(0, _CD)
        def _(j):
            pltpu.sync_copy(ones_v, deg_sh.at[idx_v.at[j]], add=True)

        plsc.subcore_barrier()
        pltpu.sync_copy(deg_sh.at[pl.ds(base, rpt)],
                        out_hbm.at[pl.ds(c * _NAGG + base, rpt)])

    return deg_kernel(idx4, zrows, ones_k)


def _sc_edge_agg(hs, src3, dst3, zrows):
    rpt = _NAGG // _NS

    @functools.partial(
        pl.kernel,
        mesh=_sc_mesh(),
        out_type=jax.ShapeDtypeStruct((_NC * _NAGG, _D), jnp.float32),
        scratch_types=[
            pltpu.VMEM_SHARED((_NAGG, _D), jnp.float32),
            pltpu.VMEM((_CE, _KE), jnp.int32),
            pltpu.VMEM((_CE, _KE), jnp.int32),
            pltpu.VMEM((_KE, _D), jnp.float32),
            pltpu.SemaphoreType.DMA,
        ],
    )
    def agg_kernel(hs_hbm, src_hbm, dst_hbm, z_hbm, out_hbm,
                   agg_sh, src_v, dst_v, rows_v, gsem):
        c = lax.axis_index("c")
        s = lax.axis_index("s")
        wid = s * _NC + c
        base = s * rpt
        pltpu.sync_copy(z_hbm, agg_sh.at[pl.ds(base, rpt)])
        pltpu.sync_copy(src_hbm.at[wid], src_v)
        pltpu.sync_copy(dst_hbm.at[wid], dst_v)
        plsc.subcore_barrier()

        @pl.loop(0, _CE)
        def _(j):
            pltpu.async_copy(hs_hbm.at[src_v.at[j]], rows_v, gsem).wait()
            pltpu.sync_copy(rows_v, agg_sh.at[dst_v.at[j]], add=True)

        plsc.subcore_barrier()
        pltpu.sync_copy(agg_sh.at[pl.ds(base, rpt)],
                        out_hbm.at[pl.ds(c * _NAGG + base, rpt)])

    return agg_kernel(hs, src3, dst3, zrows)


def _sc_decode_gather(tbl, u3, v3):
    per_w = _CQ * _KQ

    @functools.partial(
        pl.kernel,
        mesh=_sc_mesh(),
        out_type=(jax.ShapeDtypeStruct((_QP, _D), jnp.float32),
                  jax.ShapeDtypeStruct((_QP, _D), jnp.float32)),
        scratch_types=[
            pltpu.VMEM((_CQ, _KQ), jnp.int32),
            pltpu.VMEM((_CQ, _KQ), jnp.int32),
            pltpu.VMEM((_KQ, _D), jnp.float32),
            pltpu.VMEM((_KQ, _D), jnp.float32),
            pltpu.VMEM((_KQ, _D), jnp.float32),
            pltpu.VMEM((_KQ, _D), jnp.float32),
            pltpu.SemaphoreType.DMA,
            pltpu.SemaphoreType.DMA,
            pltpu.SemaphoreType.DMA,
            pltpu.SemaphoreType.DMA,
        ],
    )
    def dec_kernel(t_hbm, u_hbm, v_hbm, gu_hbm, gv_hbm,
                   u_v, v_v, ru_a, rv_a, ru_b, rv_b,
                   semu_a, semv_a, semu_b, semv_b):
        c = lax.axis_index("c")
        s = lax.axis_index("s")
        wid = s * _NC + c
        qbase = wid * per_w
        pltpu.sync_copy(u_hbm.at[wid], u_v)
        pltpu.sync_copy(v_hbm.at[wid], v_v)

        pltpu.async_copy(t_hbm.at[u_v.at[0]], ru_a, semu_a)
        pltpu.async_copy(t_hbm.at[v_v.at[0]], rv_a, semv_a)

        @pl.loop(0, _CQ // 2)
        def _(h):
            j = h * 2
            pltpu.make_async_copy(t_hbm.at[u_v.at[j]], ru_a, semu_a).wait()
            pltpu.make_async_copy(t_hbm.at[v_v.at[j]], rv_a, semv_a).wait()
            pltpu.async_copy(t_hbm.at[u_v.at[j + 1]], ru_b, semu_b)
            pltpu.async_copy(t_hbm.at[v_v.at[j + 1]], rv_b, semv_b)
            pltpu.sync_copy(ru_a, gu_hbm.at[pl.ds(qbase + j * _KQ, _KQ)])
            pltpu.sync_copy(rv_a, gv_hbm.at[pl.ds(qbase + j * _KQ, _KQ)])
            pltpu.make_async_copy(t_hbm.at[u_v.at[j + 1]], ru_b,
                                  semu_b).wait()
            pltpu.make_async_copy(t_hbm.at[v_v.at[j + 1]], rv_b,
                                  semv_b).wait()

            @pl.when(j + 2 < _CQ)
            def _():
                pltpu.async_copy(t_hbm.at[u_v.at[j + 2]], ru_a, semu_a)
                pltpu.async_copy(t_hbm.at[v_v.at[j + 2]], rv_a, semv_a)

            pltpu.sync_copy(ru_b,
                            gu_hbm.at[pl.ds(qbase + (j + 1) * _KQ, _KQ)])
            pltpu.sync_copy(rv_b,
                            gv_hbm.at[pl.ds(qbase + (j + 1) * _KQ, _KQ)])

        if _CQ % 2:
            j = _CQ - 1
            pltpu.make_async_copy(t_hbm.at[u_v.at[j]], ru_a, semu_a).wait()
            pltpu.make_async_copy(t_hbm.at[v_v.at[j]], rv_a, semv_a).wait()
            pltpu.sync_copy(ru_a, gu_hbm.at[pl.ds(qbase + j * _KQ, _KQ)])
            pltpu.sync_copy(rv_a, gv_hbm.at[pl.ds(qbase + j * _KQ, _KQ)])

    return dec_kernel(tbl, u3, v3)


_R = 400
_NB = _N // _R


def _norm_from(deg_blk):
    return lax.rsqrt(jnp.maximum(deg_blk[0, :, 0:1], 1.0))


def _prep_body(x_ref, dego_ref, o_ref):
    o_ref[...] = x_ref[...] * _norm_from(dego_ref[...])


def _tc_prep(x, degp):
    return pl.pallas_call(
        _prep_body,
        grid=(_NB,),
        in_specs=[pl.BlockSpec((_R, _D), lambda i: (i, 0)),
                  pl.BlockSpec((1, _R, _D), lambda i: (0, i, 0))],
        out_specs=pl.BlockSpec((_R, _D), lambda i: (i, 0)),
        out_shape=jax.ShapeDtypeStruct((_N, _D), jnp.float32),
    )(x, degp)


def _mid_body(p_ref, degi_ref, dego_ref, w_ref, b_ref, o_ref):
    agg = (p_ref[0] + p_ref[1]) * _norm_from(degi_ref[...])
    h = jnp.maximum(
        jnp.dot(agg, w_ref[...], preferred_element_type=jnp.float32)
        + b_ref[...], 0.0)
    o_ref[...] = h * _norm_from(dego_ref[...])


def _tc_mid(parts, degp, w1, b1):
    return pl.pallas_call(
        _mid_body,
        grid=(_NB,),
        in_specs=[pl.BlockSpec((_NC, _R, _D), lambda i: (0, i, 0)),
                  pl.BlockSpec((1, _R, _D), lambda i: (1, i, 0)),
                  pl.BlockSpec((1, _R, _D), lambda i: (0, i, 0)),
                  pl.BlockSpec((_D, _H), lambda i: (0, 0)),
                  pl.BlockSpec((1, _H), lambda i: (0, 0))],
        out_specs=pl.BlockSpec((_R, _D), lambda i: (i, 0)),
        out_shape=jax.ShapeDtypeStruct((_N, _D), jnp.float32),
    )(parts, degp, degp, w1, b1)


def _fin_body(p_ref, degi_ref, w2_ref, b2_ref, wa_ref, wb_ref, bc1_ref, t_ref):
    agg = (p_ref[0] + p_ref[1]) * _norm_from(degi_ref[...])
    z = (jnp.dot(agg, w2_ref[...], preferred_element_type=jnp.float32)
         + b2_ref[...])
    t_ref[0] = jnp.dot(z, wa_ref[...], preferred_element_type=jnp.float32)
    t_ref[1] = (jnp.dot(z, wb_ref[...], preferred_element_type=jnp.float32)
                + bc1_ref[...])


def _tc_final(parts, degp, w2, b2, wc1a, wc1b, bc1):
    return pl.pallas_call(
        _fin_body,
        grid=(_NB,),
        in_specs=[pl.BlockSpec((_NC, _R, _D), lambda i: (0, i, 0)),
                  pl.BlockSpec((1, _R, _D), lambda i: (1, i, 0)),
                  pl.BlockSpec((_D, _H), lambda i: (0, 0)),
                  pl.BlockSpec((1, _H), lambda i: (0, 0)),
                  pl.BlockSpec((_H, _H), lambda i: (0, 0)),
                  pl.BlockSpec((_H, _H), lambda i: (0, 0)),
                  pl.BlockSpec((1, _H), lambda i: (0, 0))],
        out_specs=pl.BlockSpec((2, _R, _D), lambda i: (0, i, 0)),
        out_shape=jax.ShapeDtypeStruct((2, _N, _D), jnp.float32),
    )(parts, degp, w2, b2, wc1a, wc1b, bc1)


_RQ = 2048


def _dec_body(gu_ref, gv_ref, w_ref, b_ref, o_ref):
    g = jnp.maximum(gu_ref[...] + gv_ref[...], 0.0)
    o_ref[...] = (jnp.sum(g * w_ref[...], axis=1, keepdims=True)
                  + b_ref[...])


def _tc_decode(gu, gv, wc2t, bc2):
    return pl.pallas_call(
        _dec_body,
        grid=(_QP // _RQ,),
        in_specs=[pl.BlockSpec((_RQ, _D), lambda i: (i, 0)),
                  pl.BlockSpec((_RQ, _D), lambda i: (i, 0)),
                  pl.BlockSpec((1, _D), lambda i: (0, 0)),
                  pl.BlockSpec((1, 1), lambda i: (0, 0))],
        out_specs=pl.BlockSpec((_RQ, 1), lambda i: (i, 0)),
        out_shape=jax.ShapeDtypeStruct((_QP, 1), jnp.float32),
    )(gu, gv, wc2t, bc2)


def kernel(x, edge_index, u, v, W1, b1, W2, b2, Wc1, bc1, Wc2, bc2):
    src = edge_index[0].astype(jnp.int32)
    dst = edge_index[1].astype(jnp.int32)

    didx = jnp.stack([src.reshape(_NS, _CD, _KD),
                      dst.reshape(_NS, _CD, _KD)])
    pad = _EWP - _EW
    src3 = jnp.concatenate(
        [src.reshape(_NW, _EW),
         jnp.full((_NW, pad), _N, jnp.int32)], axis=1
    ).reshape(_NW, _CE, _KE)
    dst3 = jnp.concatenate(
        [dst.reshape(_NW, _EW),
         jnp.full((_NW, pad), _NAGG - 8, jnp.int32)], axis=1
    ).reshape(_NW, _CE, _KE)
    u3 = jnp.pad(u.astype(jnp.int32), (0, _QP - _Q)).reshape(_NW, _CQ, _KQ)
    v3 = jnp.pad(v.astype(jnp.int32) + _N, (0, _QP - _Q),
                 constant_values=_N).reshape(_NW, _CQ, _KQ)

    ones_k = jnp.ones((_KD, _D), jnp.float32)
    zagg = jnp.zeros((_NAGG // _NS, _D), jnp.float32)

    degp = _sc_degrees(didx, zagg, ones_k).reshape(_NC, _NAGG, _D)

    zpad = jnp.zeros((16, _D), jnp.float32)
    hs1 = jnp.concatenate([_tc_prep(x, degp), zpad])
    parts1 = _sc_edge_agg(hs1, src3, dst3, zagg).reshape(_NC, _NAGG, _D)
    hs2 = jnp.concatenate([_tc_mid(parts1, degp, W1, b1.reshape(1, _H)), zpad])
    parts2 = _sc_edge_agg(hs2, src3, dst3, zagg).reshape(_NC, _NAGG, _D)
    tbl = _tc_final(parts2, degp, W2, b2.reshape(1, _H),
                    Wc1[:_H], Wc1[_H:], bc1.reshape(1, _H)).reshape(2 * _N, _D)

    gu, gv = _sc_decode_gather(tbl, u3, v3)
    out = _tc_decode(gu, gv, Wc2.reshape(1, _H), bc2.reshape(1, 1))
    return out.reshape(_QP)[:_Q]


# speedup vs baseline: 1.1718x; 1.0042x over previous
import functools

import jax
import jax.numpy as jnp
from jax import lax
from jax.experimental import pallas as pl
from jax.experimental.pallas import tpu as pltpu
from jax.experimental.pallas import tpu_sc as plsc

_NC = 2
_NS = 16
_NW = _NC * _NS

_N = 10000
_E = 320000
_D = 128
_H = 128
_Q = 100000

_KE = 80
_EW = _E // _NW
_CE = -(-_EW // _KE)
_EWP = _CE * _KE
_KD = 80
_CD = _E // _NS // _KD
_QP = 102400
_KQ = 128
_CQ = _QP // _NW // _KQ


def _sc_mesh():
    return plsc.VectorSubcoreMesh(core_axis_name="c", subcore_axis_name="s")


_NDEG = 20480
_NAGG = 10240


def _sc_degrees(idx4, zrows, ones_k):
    rpt = _NAGG // _NS

    @functools.partial(
        pl.kernel,
        mesh=_sc_mesh(),
        out_type=jax.ShapeDtypeStruct((_NC * _NAGG, _D), jnp.float32),
        scratch_types=[
            pltpu.VMEM_SHARED((_NAGG, _D), jnp.float32),
            pltpu.VMEM((_CD, _KD), jnp.int32),
            pltpu.VMEM((_KD, _D), jnp.float32),
        ],
    )
    def deg_kernel(idx_hbm, z_hbm, ones_hbm, out_hbm, deg_sh, idx_v, ones_v):
        c = lax.axis_index("c")
        s = lax.axis_index("s")
        base = s * rpt
        pltpu.sync_copy(z_hbm, deg_sh.at[pl.ds(base, rpt)])
        pltpu.sync_copy(idx_hbm.at[c, s], idx_v)
        pltpu.sync_copy(ones_hbm, ones_v)
        plsc.subcore_barrier()

        @pl.loop(0, _CD)
        def _(j):
            pltpu.sync_copy(ones_v, deg_sh.at[idx_v.at[j]], add=True)

        plsc.subcore_barrier()
        pltpu.sync_copy(deg_sh.at[pl.ds(base, rpt)],
                        out_hbm.at[pl.ds(c * _NAGG + base, rpt)])

    return deg_kernel(idx4, zrows, ones_k)


def _sc_edge_agg(hs, src3, dst3, zrows):
    rpt = _NAGG // _NS

    @functools.partial(
        pl.kernel,
        mesh=_sc_mesh(),
        out_type=jax.ShapeDtypeStruct((_NC * _NAGG, _D), jnp.float32),
        scratch_types=[
            pltpu.VMEM_SHARED((_NAGG, _D), jnp.float32),
            pltpu.VMEM((_CE, _KE), jnp.int32),
            pltpu.VMEM((_CE, _KE), jnp.int32),
            pltpu.VMEM((_KE, _D), jnp.float32),
            pltpu.SemaphoreType.DMA,
        ],
    )
    def agg_kernel(hs_hbm, src_hbm, dst_hbm, z_hbm, out_hbm,
                   agg_sh, src_v, dst_v, rows_v, gsem):
        c = lax.axis_index("c")
        s = lax.axis_index("s")
        wid = s * _NC + c
        base = s * rpt
        pltpu.sync_copy(z_hbm, agg_sh.at[pl.ds(base, rpt)])
        pltpu.sync_copy(src_hbm.at[wid], src_v)
        pltpu.sync_copy(dst_hbm.at[wid], dst_v)
        plsc.subcore_barrier()

        @pl.loop(0, _CE)
        def _(j):
            pltpu.async_copy(hs_hbm.at[src_v.at[j]], rows_v, gsem).wait()
            pltpu.sync_copy(rows_v, agg_sh.at[dst_v.at[j]], add=True)

        plsc.subcore_barrier()
        pltpu.sync_copy(agg_sh.at[pl.ds(base, rpt)],
                        out_hbm.at[pl.ds(c * _NAGG + base, rpt)])

    return agg_kernel(hs, src3, dst3, zrows)


def _sc_decode_gather(tbl, u3, v3):
    per_w = _CQ * _KQ

    @functools.partial(
        pl.kernel,
        mesh=_sc_mesh(),
        out_type=(jax.ShapeDtypeStruct((_QP, _D), jnp.float32),
                  jax.ShapeDtypeStruct((_QP, _D), jnp.float32)),
        scratch_types=[
            pltpu.VMEM((_CQ, _KQ), jnp.int32),
            pltpu.VMEM((_CQ, _KQ), jnp.int32),
            pltpu.VMEM((_KQ, _D), jnp.float32),
            pltpu.VMEM((_KQ, _D), jnp.float32),
            pltpu.VMEM((_KQ, _D), jnp.float32),
            pltpu.VMEM((_KQ, _D), jnp.float32),
            pltpu.SemaphoreType.DMA,
            pltpu.SemaphoreType.DMA,
            pltpu.SemaphoreType.DMA,
            pltpu.SemaphoreType.DMA,
        ],
    )
    def dec_kernel(t_hbm, u_hbm, v_hbm, gu_hbm, gv_hbm,
                   u_v, v_v, ru_a, rv_a, ru_b, rv_b,
                   semu_a, semv_a, semu_b, semv_b):
        c = lax.axis_index("c")
        s = lax.axis_index("s")
        wid = s * _NC + c
        qbase = wid * per_w
        pltpu.sync_copy(u_hbm.at[wid], u_v)
        pltpu.sync_copy(v_hbm.at[wid], v_v)

        pltpu.async_copy(t_hbm.at[u_v.at[0]], ru_a, semu_a)
        pltpu.async_copy(t_hbm.at[v_v.at[0]], rv_a, semv_a)

        @pl.loop(0, _CQ // 2)
        def _(h):
            j = h * 2
            pltpu.make_async_copy(t_hbm.at[u_v.at[j]], ru_a, semu_a).wait()
            pltpu.make_async_copy(t_hbm.at[v_v.at[j]], rv_a, semv_a).wait()
            pltpu.async_copy(t_hbm.at[u_v.at[j + 1]], ru_b, semu_b)
            pltpu.async_copy(t_hbm.at[v_v.at[j + 1]], rv_b, semv_b)
            pltpu.sync_copy(ru_a, gu_hbm.at[pl.ds(qbase + j * _KQ, _KQ)])
            pltpu.sync_copy(rv_a, gv_hbm.at[pl.ds(qbase + j * _KQ, _KQ)])
            pltpu.make_async_copy(t_hbm.at[u_v.at[j + 1]], ru_b,
                                  semu_b).wait()
            pltpu.make_async_copy(t_hbm.at[v_v.at[j + 1]], rv_b,
                                  semv_b).wait()

            @pl.when(j + 2 < _CQ)
            def _():
                pltpu.async_copy(t_hbm.at[u_v.at[j + 2]], ru_a, semu_a)
                pltpu.async_copy(t_hbm.at[v_v.at[j + 2]], rv_a, semv_a)

            pltpu.sync_copy(ru_b,
                            gu_hbm.at[pl.ds(qbase + (j + 1) * _KQ, _KQ)])
            pltpu.sync_copy(rv_b,
                            gv_hbm.at[pl.ds(qbase + (j + 1) * _KQ, _KQ)])

        if _CQ % 2:
            j = _CQ - 1
            pltpu.make_async_copy(t_hbm.at[u_v.at[j]], ru_a, semu_a).wait()
            pltpu.make_async_copy(t_hbm.at[v_v.at[j]], rv_a, semv_a).wait()
            pltpu.sync_copy(ru_a, gu_hbm.at[pl.ds(qbase + j * _KQ, _KQ)])
            pltpu.sync_copy(rv_a, gv_hbm.at[pl.ds(qbase + j * _KQ, _KQ)])

    return dec_kernel(tbl, u3, v3)


_R = 400
_NB = _N // _R


def _norm_from(deg_blk):
    return lax.rsqrt(jnp.maximum(deg_blk[0, :, 0:1], 1.0))


def _prep_body(x_ref, dego_ref, o_ref):
    o_ref[...] = x_ref[...] * _norm_from(dego_ref[...])


def _tc_prep(x, degp):
    return pl.pallas_call(
        _prep_body,
        grid=(_NB,),
        in_specs=[pl.BlockSpec((_R, _D), lambda i: (i, 0)),
                  pl.BlockSpec((1, _R, _D), lambda i: (0, i, 0))],
        out_specs=pl.BlockSpec((_R, _D), lambda i: (i, 0)),
        out_shape=jax.ShapeDtypeStruct((_N, _D), jnp.float32),
    )(x, degp)


def _mid_body(p_ref, degi_ref, dego_ref, w_ref, b_ref, o_ref):
    agg = (p_ref[0] + p_ref[1]) * _norm_from(degi_ref[...])
    h = jnp.maximum(
        jnp.dot(agg, w_ref[...], preferred_element_type=jnp.float32)
        + b_ref[...], 0.0)
    o_ref[...] = h * _norm_from(dego_ref[...])


def _tc_mid(parts, degp, w1, b1):
    return pl.pallas_call(
        _mid_body,
        grid=(_NB,),
        in_specs=[pl.BlockSpec((_NC, _R, _D), lambda i: (0, i, 0)),
                  pl.BlockSpec((1, _R, _D), lambda i: (1, i, 0)),
                  pl.BlockSpec((1, _R, _D), lambda i: (0, i, 0)),
                  pl.BlockSpec((_D, _H), lambda i: (0, 0)),
                  pl.BlockSpec((1, _H), lambda i: (0, 0))],
        out_specs=pl.BlockSpec((_R, _D), lambda i: (i, 0)),
        out_shape=jax.ShapeDtypeStruct((_N, _D), jnp.float32),
    )(parts, degp, degp, w1, b1)


def _fin_body(p_ref, degi_ref, w2_ref, b2_ref, wa_ref, wb_ref, bc1_ref, t_ref):
    agg = (p_ref[0] + p_ref[1]) * _norm_from(degi_ref[...])
    z = (jnp.dot(agg, w2_ref[...], preferred_element_type=jnp.float32)
         + b2_ref[...])
    t_ref[0] = jnp.dot(z, wa_ref[...], preferred_element_type=jnp.float32)
    t_ref[1] = (jnp.dot(z, wb_ref[...], preferred_element_type=jnp.float32)
                + bc1_ref[...])


def _tc_final(parts, degp, w2, b2, wc1a, wc1b, bc1):
    return pl.pallas_call(
        _fin_body,
        grid=(_NB,),
        in_specs=[pl.BlockSpec((_NC, _R, _D), lambda i: (0, i, 0)),
                  pl.BlockSpec((1, _R, _D), lambda i: (1, i, 0)),
                  pl.BlockSpec((_D, _H), lambda i: (0, 0)),
                  pl.BlockSpec((1, _H), lambda i: (0, 0)),
                  pl.BlockSpec((_H, _H), lambda i: (0, 0)),
                  pl.BlockSpec((_H, _H), lambda i: (0, 0)),
                  pl.BlockSpec((1, _H), lambda i: (0, 0))],
        out_specs=pl.BlockSpec((2, _R, _D), lambda i: (0, i, 0)),
        out_shape=jax.ShapeDtypeStruct((2, _N, _D), jnp.float32),
    )(parts, degp, w2, b2, wc1a, wc1b, bc1)


_RQ = 2048


def _dec_body(gu_ref, gv_ref, w_ref, b_ref, o_ref):
    g = jnp.maximum(gu_ref[...] + gv_ref[...], 0.0)
    o_ref[...] = (jnp.sum(g * w_ref[...], axis=1, keepdims=True)
                  + b_ref[...])


def _tc_decode(gu, gv, wc2t, bc2):
    return pl.pallas_call(
        _dec_body,
        grid=(_QP // _RQ,),
        in_specs=[pl.BlockSpec((_RQ, _D), lambda i: (i, 0)),
                  pl.BlockSpec((_RQ, _D), lambda i: (i, 0)),
                  pl.BlockSpec((1, _D), lambda i: (0, 0)),
                  pl.BlockSpec((1, 1), lambda i: (0, 0))],
        out_specs=pl.BlockSpec((_RQ, 1), lambda i: (i, 0)),
        out_shape=jax.ShapeDtypeStruct((_QP, 1), jnp.float32),
    )(gu, gv, wc2t, bc2)


def kernel(x, edge_index, u, v, W1, b1, W2, b2, Wc1, bc1, Wc2, bc2):
    src = edge_index[0].astype(jnp.int32)
    dst = edge_index[1].astype(jnp.int32)

    didx = jnp.stack([src.reshape(_NS, _CD, _KD),
                      dst.reshape(_NS, _CD, _KD)])
    pad = _EWP - _EW
    src3 = jnp.concatenate(
        [src.reshape(_NW, _EW),
         jnp.full((_NW, pad), _N, jnp.int32)], axis=1
    ).reshape(_NW, _CE, _KE)
    dst3 = jnp.concatenate(
        [dst.reshape(_NW, _EW),
         jnp.full((_NW, pad), _NAGG - 8, jnp.int32)], axis=1
    ).reshape(_NW, _CE, _KE)
    u3 = jnp.pad(u.astype(jnp.int32), (0, _QP - _Q)).reshape(_NW, _CQ, _KQ)
    v3 = jnp.pad(v.astype(jnp.int32) + _N, (0, _QP - _Q),
                 constant_values=_N).reshape(_NW, _CQ, _KQ)

    ones_k = jnp.ones((_KD, _D), jnp.float32)
    zagg = jnp.zeros((_NAGG // _NS, _D), jnp.float32)

    degp = _sc_degrees(didx, zagg, ones_k).reshape(_NC, _NAGG, _D)

    zpad = jnp.zeros((16, _D), jnp.float32)
    hs1 = jnp.concatenate([_tc_prep(x, degp), zpad])
    parts1 = _sc_edge_agg(hs1, src3, dst3, zagg).reshape(_NC, _NAGG, _D)
    hs2 = jnp.concatenate([_tc_mid(parts1, degp, W1, b1.reshape(1, _H)), zpad])
    parts2 = _sc_edge_agg(hs2, src3, dst3, zagg).reshape(_NC, _NAGG, _D)
    tbl = _tc_final(parts2, degp, W2, b2.reshape(1, _H),
                    Wc1[:_H], Wc1[_H:], bc1.reshape(1, _H)).reshape(2 * _N, _D)

    gu, gv = _sc_decode_gather(tbl, u3, v3)
    out = _tc_decode(gu, gv, Wc2.reshape(1, _H), bc2.reshape(1, 1))
    return out.reshape(_QP)[:_Q]
